# Initial kernel scaffold; baseline (speedup 1.0000x reference)
import jax
import jax.numpy as jnp
from jax.experimental import pallas as pl


def kernel(x_q, x, fps_idx, anchor_points, map_idx, anchor_dist, W1, W2, W3):
    raise NotImplementedError("write your pallas kernel here")


# speedup vs baseline: 19.5797x; 19.5797x over previous
import functools

import jax
import jax.numpy as jnp
from jax import lax
from jax.experimental import pallas as pl
from jax.experimental.pallas import tpu as pltpu
from jax.experimental.pallas import tpu_sc as plsc

B, N, M, K = 4, 4096, 256, 16
RB = 256
NB = 512
NW = 32
N_PER_W = (B * N) // NW
CH = 256


def _topk_body(xq_ref, xt_ref, idx_ref):
    xq = xq_ref[0]
    xt = xt_ref[0]
    dot = jnp.dot(xq.astype(jnp.bfloat16), xt.astype(jnp.bfloat16),
                  preferred_element_type=jnp.float32)
    q0, q1, q2 = xq[:, 0:1], xq[:, 1:2], xq[:, 2:3]
    xq2 = (q0 * q0 + q1 * q1) + q2 * q2
    x0, x1, x2c = xt[0:1, :], xt[1:2, :], xt[2:3, :]
    x2 = (x0 * x0 + x1 * x1) + x2c * x2c
    d = (jnp.float32(-2.0) * dot + xq2) + x2
    colf = lax.broadcasted_iota(jnp.int32, (RB, N), 1).astype(jnp.float32)
    big = jnp.float32(3.0e38)
    cols = []
    for _ in range(K):
        m = jnp.min(d, axis=1, keepdims=True)
        sel = jnp.where(d == m, colf, big)
        a = jnp.min(sel, axis=1, keepdims=True)
        cols.append(a)
        d = jnp.where(colf == a, big, d)
    idx_ref[0] = jnp.concatenate(cols, axis=1).astype(jnp.int32)


def _knn_idx(x_q, x):
    xt = jnp.transpose(x, (0, 2, 1))
    return pl.pallas_call(
        _topk_body,
        grid=(B, N // RB),
        in_specs=[
            pl.BlockSpec((1, RB, 3), lambda b, i: (b, i, 0)),
            pl.BlockSpec((1, 3, N), lambda b, i: (b, 0, 0)),
        ],
        out_specs=pl.BlockSpec((1, RB, K), lambda b, i: (b, i, 0)),
        out_shape=jax.ShapeDtypeStruct((B, N, K), jnp.int32),
    )(x_q, xt)


def _sqrt16(a):
    a = jnp.maximum(a, jnp.float32(1e-30))
    i = lax.bitcast_convert_type(a, jnp.int32)
    i = jnp.int32(0x5F3759DF) - (i >> 1)
    y = lax.bitcast_convert_type(i, jnp.float32)
    for _ in range(3):
        y = y * (jnp.float32(1.5) - jnp.float32(0.5) * a * y * y)
    return a * y


def _full16(v):
    return jnp.broadcast_to(v, (16,))


def _sc_feat_body(xq_hbm, x_hbm, map_hbm, anch_hbm, ad_hbm, idx_hbm, out_hbm,
                  x_v, xq_v, map_v, anch_v, ad_v, idx_v, da_v, feat_v):
    wid = lax.axis_index("s") * 2 + lax.axis_index("c")
    b = wid // (NW // B)
    n_base = (wid * N_PER_W) % N

    pltpu.sync_copy(x_hbm.at[b], x_v)
    pltpu.sync_copy(map_hbm.at[b], map_v)
    pltpu.sync_copy(anch_hbm.at[b], anch_v)
    pltpu.sync_copy(ad_hbm.at[b], ad_v)

    lanes = lax.iota(jnp.int32, 16)

    for ci in range(N_PER_W // CH):
        n0 = n_base + ci * CH
        pltpu.sync_copy(idx_hbm.at[b, pl.ds(n0, CH)], idx_v)
        pltpu.sync_copy(xq_hbm.at[b, pl.ds(n0 * 3, CH * 3)], xq_v)

        def da_body(g, _):
            rows = lanes + g * 16
            a16 = plsc.load_gather(map_v, [rows + n0])
            acc = None
            for c in range(3):
                xqc = plsc.load_gather(xq_v, [rows * 3 + c])
                aac = plsc.load_gather(anch_v, [a16 * 3 + c])
                t = xqc - aac
                acc = t * t if acc is None else acc + t * t
            da_v[pl.ds(g * 16, 16)] = _sqrt16(acc)
            return _

        lax.fori_loop(0, CH // 16, da_body, None)

        def n_body(n, _):
            idxk = idx_v[n, :]
            nb = []
            r1 = []
            for c in range(3):
                nbc = plsc.load_gather(x_v, [idxk * 3 + c])
                nb.append(nbc)
                xqc = plsc.load_gather(xq_v, [_full16(n * 3 + c)])
                r1.append(jnp.abs(xqc - nbc))
            bk = plsc.load_gather(map_v, [idxk])
            acc = None
            for c in range(3):
                abc = plsc.load_gather(anch_v, [bk * 3 + c])
                t = nb[c] - abc
                acc = t * t if acc is None else acc + t * t
            dnb = _sqrt16(acc)
            a16 = plsc.load_gather(map_v, [_full16(n + n0)])
            adab = plsc.load_gather(ad_v, [a16, bk])
            da16 = plsc.load_gather(da_v, [_full16(n)])
            r2 = da16 + adab + dnb
            feat_v[0, pl.ds(n * K, K)] = r1[0]
            feat_v[1, pl.ds(n * K, K)] = r1[1]
            feat_v[2, pl.ds(n * K, K)] = r1[2]
            feat_v[3, pl.ds(n * K, K)] = r2
            feat_v[4, pl.ds(n * K, K)] = r2
            feat_v[5, pl.ds(n * K, K)] = r2
            return _

        lax.fori_loop(0, CH, n_body, None)

        for c in range(6):
            pltpu.sync_copy(feat_v.at[c], out_hbm.at[b, c, pl.ds(n0 * K, CH * K)])


def _sc_feat(x_q, x, map_idx, anchor_points, anchor_dist, idx):
    mesh = plsc.VectorSubcoreMesh(core_axis_name="c", subcore_axis_name="s")
    fn = pl.kernel(
        _sc_feat_body, mesh=mesh,
        compiler_params=pltpu.CompilerParams(
            needs_layout_passes=False, use_tc_tiling_on_sc=False),
        out_type=jax.ShapeDtypeStruct((B, 6, N * K), jnp.float32),
        scratch_types=[
            pltpu.VMEM((N * 3,), jnp.float32),
            pltpu.VMEM((CH * 3,), jnp.float32),
            pltpu.VMEM((N,), jnp.int32),
            pltpu.VMEM((M * 3,), jnp.float32),
            pltpu.VMEM((M, M), jnp.float32),
            pltpu.VMEM((CH, K), jnp.int32),
            pltpu.VMEM((CH,), jnp.float32),
            pltpu.VMEM((6, CH * K), jnp.float32),
        ],
    )
    return fn(x_q.reshape(B, N * 3), x.reshape(B, N * 3), map_idx,
              anchor_points.reshape(B, M * 3), anchor_dist, idx)


def _mlp_body(feat_ref, w1_ref, w2_ref, w3_ref, wout_ref, lf_ref):
    f = feat_ref[0]
    h = jnp.maximum(jnp.dot(w1_ref[...], f, preferred_element_type=jnp.float32), 0.0)
    h = jnp.maximum(jnp.dot(w2_ref[...], h, preferred_element_type=jnp.float32), 0.0)
    w = jnp.dot(w3_ref[...], h, preferred_element_type=jnp.float32)
    wout_ref[0] = w
    m = w
    s = 1
    while s < K:
        m = jnp.maximum(m, jnp.concatenate([m[:, s:], m[:, :s]], axis=1))
        s *= 2
    rows = lax.broadcasted_iota(jnp.int32, (NB * K, NB), 0)
    cols = lax.broadcasted_iota(jnp.int32, (NB * K, NB), 1)
    sel = (rows == cols * K).astype(jnp.bfloat16)
    hi = m.astype(jnp.bfloat16)
    lo = (m - hi.astype(jnp.float32)).astype(jnp.bfloat16)
    lf = (jnp.dot(hi, sel, preferred_element_type=jnp.float32)
          + jnp.dot(lo, sel, preferred_element_type=jnp.float32))
    lf_ref[0] = lf


def _mlp(feat, W1, W2, W3):
    return pl.pallas_call(
        _mlp_body,
        grid=(B, N // NB),
        in_specs=[
            pl.BlockSpec((1, 6, NB * K), lambda b, i: (b, 0, i)),
            pl.BlockSpec((32, 6), lambda b, i: (0, 0)),
            pl.BlockSpec((64, 32), lambda b, i: (0, 0)),
            pl.BlockSpec((128, 64), lambda b, i: (0, 0)),
        ],
        out_specs=[
            pl.BlockSpec((1, 128, NB * K), lambda b, i: (b, 0, i)),
            pl.BlockSpec((1, 128, NB), lambda b, i: (b, 0, i)),
        ],
        out_shape=[
            jax.ShapeDtypeStruct((B, 128, N * K), jnp.float32),
            jax.ShapeDtypeStruct((B, 128, N), jnp.float32),
        ],
    )(feat, W1, W2, W3)


def kernel(x_q, x, fps_idx, anchor_points, map_idx, anchor_dist, W1, W2, W3):
    idx = _knn_idx(x_q, x)
    feat = _sc_feat(x_q, x, map_idx, anchor_points, anchor_dist, idx)
    wflat, lf = _mlp(feat, W1, W2, W3)
    return (jnp.transpose(lf, (0, 2, 1)), wflat.reshape(B, 128, N, K))
